# Initial kernel scaffold; baseline (speedup 1.0000x reference)
import jax
import jax.numpy as jnp
from jax.experimental import pallas as pl


def kernel(x, edge_index, W1_l, b1_l, W1_r, W2_l, b2_l, W2_r):
    raise NotImplementedError("write your pallas kernel here")


# speedup vs baseline: 5.1717x; 5.1717x over previous
import functools

import jax
import jax.numpy as jnp
from jax import lax
from jax.experimental import pallas as pl
from jax.experimental.pallas import tpu as pltpu
from jax.experimental.pallas import tpu_sc as plsc

N = 10000
E = 320000
D = 128

NC = 2
NS = 16
NW = NC * NS
EPW = E // NW
C = 80
NCHUNK = EPW // C
NP = 10240
RPT = NP // NS


def _sc_agg(x, src, dst, with_cnt):
    mesh = plsc.VectorSubcoreMesh(core_axis_name="c", subcore_axis_name="s")

    out_type = [jax.ShapeDtypeStruct((NC, NP, D), jnp.float32)]
    scratch = [
        pltpu.VMEM((C,), jnp.int32),
        pltpu.VMEM((C,), jnp.int32),
        pltpu.VMEM((C, D), jnp.float32),
        pltpu.VMEM_SHARED((NP, D), jnp.float32),
        pltpu.SemaphoreType.DMA,
    ]
    if with_cnt:
        out_type.append(jax.ShapeDtypeStruct((NW * NP,), jnp.float32))
        scratch.append(pltpu.VMEM((NP,), jnp.float32))

    @functools.partial(
        pl.kernel, mesh=mesh, out_type=out_type, scratch_types=scratch,
        compiler_params=pltpu.CompilerParams(needs_layout_passes=False))
    def k(x_hbm, src_hbm, dst_hbm, *refs):
        if with_cnt:
            agg_out, cnt_out, src_v, dst_v, rows_v, acc, sem, cnt_t = refs
        else:
            agg_out, src_v, dst_v, rows_v, acc, sem = refs
        cid = lax.axis_index("c")
        sid = lax.axis_index("s")
        wid = sid * NC + cid

        def zb(i, carry):
            for j in range(D // 16):
                rows_v[i, pl.ds(j * 16, 16)] = jnp.zeros((16,), jnp.float32)
            return carry
        lax.fori_loop(0, C, zb, 0)
        for t in range(RPT // C):
            pltpu.sync_copy(rows_v, acc.at[pl.ds(sid * RPT + t * C, C)])
        if with_cnt:
            def zc(i, carry):
                cnt_t[pl.ds(i * 16, 16)] = jnp.zeros((16,), jnp.float32)
                return carry
            lax.fori_loop(0, NP // 16, zc, 0)
        plsc.subcore_barrier()

        def body(i, carry):
            base = wid * EPW + i * C
            pltpu.sync_copy(src_hbm.at[pl.ds(base, C)], src_v)
            pltpu.sync_copy(dst_hbm.at[pl.ds(base, C)], dst_v)
            pltpu.async_copy(x_hbm.at[src_v], rows_v, sem).wait()
            pltpu.sync_copy(rows_v, acc.at[dst_v], add=True)
            if with_cnt:
                ones16 = jnp.ones((16,), jnp.float32)
                for k in range(C // 16):
                    idx = dst_v[pl.ds(k * 16, 16)]
                    plsc.addupdate_scatter(cnt_t, [idx], ones16)
            return carry
        lax.fori_loop(0, NCHUNK, body, 0)
        plsc.subcore_barrier()

        pltpu.sync_copy(acc.at[pl.ds(sid * RPT, RPT)],
                        agg_out.at[cid, pl.ds(sid * RPT, RPT)])
        if with_cnt:
            pltpu.sync_copy(cnt_t, cnt_out.at[pl.ds(wid * NP, NP)])

    return k(x, src, dst)


def _combine_body(p_ref, c_ref, x_ref, wl_ref, b_ref, wr_ref, o_ref, *, relu):
    cnt = jnp.maximum(jnp.sum(c_ref[:], axis=0), 1.0)[:, None]
    mean = (p_ref[0] + p_ref[1]) / cnt
    dn = (((1,), (1,)), ((), ()))
    y = lax.dot_general(mean, wl_ref[:], dn,
                        preferred_element_type=jnp.float32)
    y = y + b_ref[:]
    y = y + lax.dot_general(x_ref[:], wr_ref[:], dn,
                            preferred_element_type=jnp.float32)
    o_ref[:] = jnp.maximum(y, 0.0) if relu else y


def _tc_combine(p, c, x, W_l, b_l, W_r, relu):
    R = 1024
    grid = (NP // R,)
    return pl.pallas_call(
        functools.partial(_combine_body, relu=relu),
        grid=grid,
        in_specs=[
            pl.BlockSpec((NC, R, D), lambda i: (0, i, 0)),
            pl.BlockSpec((NW, R), lambda i: (0, i)),
            pl.BlockSpec((R, D), lambda i: (i, 0)),
            pl.BlockSpec((D, D), lambda i: (0, 0)),
            pl.BlockSpec((1, D), lambda i: (0, 0)),
            pl.BlockSpec((D, D), lambda i: (0, 0)),
        ],
        out_specs=pl.BlockSpec((R, D), lambda i: (i, 0)),
        out_shape=jax.ShapeDtypeStruct((NP, D), jnp.float32),
    )(p, c.reshape(NW, NP), x, W_l, b_l.reshape(1, D), W_r)


def kernel(x, edge_index, W1_l, b1_l, W1_r, W2_l, b2_l, W2_r):
    src = edge_index[0]
    dst = edge_index[1]
    x_p = jnp.pad(x, ((0, NP - N), (0, 0)))
    agg1, cnt = _sc_agg(x, src, dst, with_cnt=True)
    h = _tc_combine(agg1, cnt, x_p, W1_l, b1_l, W1_r, relu=True)
    (agg2,) = _sc_agg(h, src, dst, with_cnt=False)
    out = _tc_combine(agg2, cnt, h, W2_l, b2_l, W2_r, relu=False)
    return out[:N]
